# baseline (speedup 1.0000x reference)
import jax
import jax.numpy as jnp
from jax import lax
from jax.experimental import pallas as pl

NEIGHBOUR_IOU = 0.2

_N_DT = 2000
_N_GT = 5000
_BR = 200
_RG = 8


def _tc_kernel(dt_ref, cat_ref, dtgt_ref, dtdt_ref, mask_ref):
    cx1 = cat_ref[0:1, :]
    cy1 = cat_ref[1:2, :]
    cx2 = cat_ref[2:3, :]
    cy2 = cat_ref[3:4, :]
    car = (cx2 - cx1) * (cy2 - cy1)

    dx1 = cx1[:, :_N_DT]
    dy1 = cy1[:, :_N_DT]
    dx2 = cx2[:, :_N_DT]
    dy2 = cy2[:, :_N_DT]
    dar = car[:, :_N_DT]
    gx1 = cx1[:, _N_DT:_N_DT + _N_GT]
    gy1 = cy1[:, _N_DT:_N_DT + _N_GT]
    gx2 = cx2[:, _N_DT:_N_DT + _N_GT]
    gy2 = cy2[:, _N_DT:_N_DT + _N_GT]
    gar = car[:, _N_DT:_N_DT + _N_GT]

    def group(g, _):
        rs = pl.ds(g * _RG, _RG)
        d = dt_ref[rs, :]
        x1r = d[:, 0:1]
        y1r = d[:, 1:2]
        x2r = d[:, 2:3]
        y2r = d[:, 3:4]
        ar = (x2r - x1r) * (y2r - y1r)

        def strip(x1c, y1c, x2c, y2c, ac):
            ix1 = jnp.maximum(x1r, x1c)
            iy1 = jnp.maximum(y1r, y1c)
            ix2 = jnp.minimum(x2r, x2c)
            iy2 = jnp.minimum(y2r, y2c)
            inter = (jnp.maximum(ix2 - ix1, 0.0)
                     * jnp.maximum(iy2 - iy1, 0.0))
            union = ar + ac - inter
            return inter, union

        ig, ug = strip(gx1, gy1, gx2, gy2, gar)
        dtgt_ref[rs, :] = ig * pl.reciprocal(ug, approx=True)
        idd, udd = strip(dx1, dy1, dx2, dy2, dar)
        dtdt_ref[rs, :] = idd * pl.reciprocal(udd, approx=True)
        mask_ref[rs, :] = idd >= NEIGHBOUR_IOU * udd
        return 0

    lax.fori_loop(0, _BR // _RG, group, 0)


def kernel(detections, gt_boxes):
    dt = detections[:_N_DT]
    cat = jnp.concatenate([dt, gt_boxes], axis=0).T
    cat = jnp.pad(cat, ((0, 4), (0, 7040 - _N_DT - _N_GT)))

    dtgt, dtdt, mask = pl.pallas_call(
        _tc_kernel,
        grid=(_N_DT // _BR,),
        in_specs=[
            pl.BlockSpec((_BR, 4), lambda i: (i, 0)),
            pl.BlockSpec((8, 7040), lambda i: (0, 0)),
        ],
        out_specs=[
            pl.BlockSpec((_BR, _N_GT), lambda i: (i, 0)),
            pl.BlockSpec((_BR, _N_DT), lambda i: (i, 0)),
            pl.BlockSpec((_BR, _N_DT), lambda i: (i, 0)),
        ],
        out_shape=[
            jax.ShapeDtypeStruct((_N_DT, _N_GT), jnp.float32),
            jax.ShapeDtypeStruct((_N_DT, _N_DT), jnp.float32),
            jax.ShapeDtypeStruct((_N_DT, _N_DT), jnp.bool_),
        ],
    )(dt, cat)
    return dtgt, dtdt, mask


# speedup vs baseline: 2.2308x; 2.2308x over previous
import jax
import jax.numpy as jnp
from jax import lax
from jax.experimental import pallas as pl

NEIGHBOUR_IOU = 0.2

_N_DT = 2000
_N_GT = 5000
_BR = 200
_RG = 8


def _tc_kernel(dt_ref, cat_ref, dtgt_ref, dtdt_ref, mask_ref):
    cx1 = cat_ref[0:1, :]
    cy1 = cat_ref[1:2, :]
    cx2 = cat_ref[2:3, :]
    cy2 = cat_ref[3:4, :]
    car = (cx2 - cx1) * (cy2 - cy1)

    dx1 = cx1[:, :_N_DT]
    dy1 = cy1[:, :_N_DT]
    dx2 = cx2[:, :_N_DT]
    dy2 = cy2[:, :_N_DT]
    dar = car[:, :_N_DT]
    gx1 = cx1[:, _N_DT:_N_DT + _N_GT]
    gy1 = cy1[:, _N_DT:_N_DT + _N_GT]
    gx2 = cx2[:, _N_DT:_N_DT + _N_GT]
    gy2 = cy2[:, _N_DT:_N_DT + _N_GT]
    gar = car[:, _N_DT:_N_DT + _N_GT]

    d = dt_ref[...]
    x1r = d[:, 0:1]
    y1r = d[:, 1:2]
    x2r = d[:, 2:3]
    y2r = d[:, 3:4]
    ar = (x2r - x1r) * (y2r - y1r)

    def strip(x1c, y1c, x2c, y2c, ac):
        ix1 = jnp.maximum(x1r, x1c)
        iy1 = jnp.maximum(y1r, y1c)
        ix2 = jnp.minimum(x2r, x2c)
        iy2 = jnp.minimum(y2r, y2c)
        inter = (jnp.maximum(ix2 - ix1, 0.0)
                 * jnp.maximum(iy2 - iy1, 0.0))
        union = ar + ac - inter
        return inter, union

    ig, ug = strip(gx1, gy1, gx2, gy2, gar)
    dtgt_ref[...] = ig * pl.reciprocal(ug, approx=True)
    idd, udd = strip(dx1, dy1, dx2, dy2, dar)
    dtdt_ref[...] = idd * pl.reciprocal(udd, approx=True)
    mask_ref[...] = idd >= NEIGHBOUR_IOU * udd


def kernel(detections, gt_boxes):
    dt = detections[:_N_DT]
    cat = jnp.concatenate([dt, gt_boxes], axis=0).T
    cat = jnp.pad(cat, ((0, 4), (0, 7040 - _N_DT - _N_GT)))

    dtgt, dtdt, mask = pl.pallas_call(
        _tc_kernel,
        grid=(_N_DT // _BR,),
        in_specs=[
            pl.BlockSpec((_BR, 4), lambda i: (i, 0)),
            pl.BlockSpec((8, 7040), lambda i: (0, 0)),
        ],
        out_specs=[
            pl.BlockSpec((_BR, _N_GT), lambda i: (i, 0)),
            pl.BlockSpec((_BR, _N_DT), lambda i: (i, 0)),
            pl.BlockSpec((_BR, _N_DT), lambda i: (i, 0)),
        ],
        out_shape=[
            jax.ShapeDtypeStruct((_N_DT, _N_GT), jnp.float32),
            jax.ShapeDtypeStruct((_N_DT, _N_DT), jnp.float32),
            jax.ShapeDtypeStruct((_N_DT, _N_DT), jnp.bool_),
        ],
    )(dt, cat)
    return dtgt, dtdt, mask
